# Initial kernel scaffold; baseline (speedup 1.0000x reference)
import jax
import jax.numpy as jnp
from jax.experimental import pallas as pl


def kernel(inputs, targets, weight):
    raise NotImplementedError("write your pallas kernel here")


# speedup vs baseline: 1.1390x; 1.1390x over previous
import functools

import jax
import jax.numpy as jnp
from jax import lax
from jax.experimental import pallas as pl
from jax.experimental.pallas import tpu as pltpu
from jax.experimental.pallas import tpu_sc as plsc

NROWS, NCOLS = 4096, 2048
N_ELEMS = NROWS * NCOLS
NC, NS, L = 2, 16, 16
NW = NC * NS
ROWS_PER_W = NROWS // NW
RPC = 8
NCH = ROWS_PER_W // RPC
NBUF = 2
NGRP = NCH // NBUF
CSTEPS = NCOLS // L

_mesh = plsc.VectorSubcoreMesh(core_axis_name="c", subcore_axis_name="s")


@functools.partial(
    pl.kernel,
    mesh=_mesh,
    out_type=jax.ShapeDtypeStruct((NW, L), jnp.float32),
    scratch_types=[
        pltpu.VMEM((NBUF, RPC, NCOLS), jnp.float32),
        pltpu.VMEM((NBUF, RPC, NCOLS), jnp.int32),
        pltpu.VMEM((L,), jnp.float32),
        pltpu.VMEM((L,), jnp.float32),
        pltpu.SemaphoreType.DMA,
        pltpu.SemaphoreType.DMA,
        pltpu.SemaphoreType.DMA,
        pltpu.SemaphoreType.DMA,
    ],
)
def _wmse_sc(x_hbm, t_hbm, w_hbm, out_hbm, xbuf, tbuf, wv, accv, *sems):
    semx = sems[:NBUF]
    semt = sems[NBUF:]
    wid = lax.axis_index("s") * NC + lax.axis_index("c")
    base_row = wid * ROWS_PER_W

    pltpu.sync_copy(w_hbm, wv)

    def start(c, b):
        row = base_row + c * RPC
        pltpu.async_copy(x_hbm.at[pl.ds(row, RPC)], xbuf.at[b], semx[b])
        pltpu.async_copy(t_hbm.at[pl.ds(row, RPC)], tbuf.at[b], semt[b])

    def wait(b):
        pltpu.make_async_copy(x_hbm.at[pl.ds(0, RPC)], xbuf.at[b], semx[b]).wait()
        pltpu.make_async_copy(t_hbm.at[pl.ds(0, RPC)], tbuf.at[b], semt[b]).wait()

    for b in range(NBUF):
        start(b, b)

    wtab = wv[...]

    def chunk_compute(b, accs):
        def step(i, accs):
            o = i * L
            accs = list(accs)
            for r in range(RPC):
                t = tbuf[b, r, pl.ds(o, L)]
                x = xbuf[b, r, pl.ds(o, L)]
                w = wtab.at[t].get(mode="promise_in_bounds")
                d = x - t.astype(jnp.float32)
                accs[r % 4] = accs[r % 4] + (w * d) * d
            return tuple(accs)

        return lax.fori_loop(0, CSTEPS, step, accs)

    def group(g, accs):
        for b in range(NBUF):
            c = g * NBUF + b
            wait(b)
            accs = chunk_compute(b, accs)
            start(c + NBUF, b)
        return accs

    zero = jnp.zeros((L,), jnp.float32)
    accs = (zero, zero, zero, zero)
    accs = lax.fori_loop(0, NGRP - 1, group, accs)
    for b in range(NBUF):
        wait(b)
        accs = chunk_compute(b, accs)

    accv[...] = (accs[0] + accs[1]) + (accs[2] + accs[3])
    pltpu.sync_copy(accv, out_hbm.at[wid])


def kernel(inputs, targets, weight):
    partials = _wmse_sc(inputs, targets, weight)
    return jnp.sum(partials) / N_ELEMS


# speedup vs baseline: 1.2623x; 1.1082x over previous
import functools

import jax
import jax.numpy as jnp
from jax import lax
from jax.experimental import pallas as pl
from jax.experimental.pallas import tpu as pltpu
from jax.experimental.pallas import tpu_sc as plsc

NROWS, NCOLS = 4096, 2048
N_ELEMS = NROWS * NCOLS
NC, NS, L = 2, 16, 16
NW = NC * NS

SC_ROWS = 2560
TC_ROWS = NROWS - SC_ROWS
TC_BLOCK = 128

ROWS_PER_W = SC_ROWS // NW
RPC = 8
NCH = ROWS_PER_W // RPC
NBUF = 2
NGRP = NCH // NBUF
CSTEPS = NCOLS // L

_mesh = plsc.VectorSubcoreMesh(core_axis_name="c", subcore_axis_name="s")


@functools.partial(
    pl.kernel,
    mesh=_mesh,
    out_type=jax.ShapeDtypeStruct((NW, L), jnp.float32),
    scratch_types=[
        pltpu.VMEM((NBUF, RPC, NCOLS), jnp.float32),
        pltpu.VMEM((NBUF, RPC, NCOLS), jnp.int32),
        pltpu.VMEM((L,), jnp.float32),
        pltpu.VMEM((L,), jnp.float32),
        pltpu.SemaphoreType.DMA,
        pltpu.SemaphoreType.DMA,
        pltpu.SemaphoreType.DMA,
        pltpu.SemaphoreType.DMA,
    ],
)
def _wmse_sc(x_hbm, t_hbm, w_hbm, out_hbm, xbuf, tbuf, wv, accv, *sems):
    semx = sems[:NBUF]
    semt = sems[NBUF:]
    wid = lax.axis_index("s") * NC + lax.axis_index("c")
    base_row = wid * ROWS_PER_W

    pltpu.sync_copy(w_hbm, wv)

    def start(c, b):
        row = base_row + c * RPC
        pltpu.async_copy(x_hbm.at[pl.ds(row, RPC)], xbuf.at[b], semx[b])
        pltpu.async_copy(t_hbm.at[pl.ds(row, RPC)], tbuf.at[b], semt[b])

    def wait(b):
        pltpu.make_async_copy(x_hbm.at[pl.ds(0, RPC)], xbuf.at[b], semx[b]).wait()
        pltpu.make_async_copy(t_hbm.at[pl.ds(0, RPC)], tbuf.at[b], semt[b]).wait()

    for b in range(NBUF):
        start(b, b)

    wtab = wv[...]

    def chunk_compute(b, accs):
        def step(i, accs):
            o = i * L
            accs = list(accs)
            for r in range(RPC):
                t = tbuf[b, r, pl.ds(o, L)]
                x = xbuf[b, r, pl.ds(o, L)]
                w = wtab.at[t].get(mode="promise_in_bounds")
                d = x - t.astype(jnp.float32)
                accs[r % 4] = accs[r % 4] + (w * d) * d
            return tuple(accs)

        return lax.fori_loop(0, CSTEPS, step, accs)

    def group(g, accs):
        for b in range(NBUF):
            c = g * NBUF + b
            wait(b)
            accs = chunk_compute(b, accs)
            start(c + NBUF, b)
        return accs

    zero = jnp.zeros((L,), jnp.float32)
    accs = (zero, zero, zero, zero)
    accs = lax.fori_loop(0, NGRP - 1, group, accs)
    for b in range(NBUF):
        wait(b)
        accs = chunk_compute(b, accs)

    accv[...] = (accs[0] + accs[1]) + (accs[2] + accs[3])
    pltpu.sync_copy(accv, out_hbm.at[wid])


def _wmse_tc_body(x_ref, t_ref, w_ref, out_ref, acc_ref):
    i = pl.program_id(0)

    @pl.when(i == 0)
    def _():
        acc_ref[0, 0] = 0.0

    t = t_ref[...]
    x = x_ref[...]
    d = x - t.astype(jnp.float32)
    s = d * d
    w = jnp.full(t.shape, w_ref[0], jnp.float32)
    for k in range(1, 16):
        w = jnp.where(t == k, w_ref[k], w)
    acc_ref[0, 0] += jnp.sum(w * s)

    @pl.when(i == TC_ROWS // TC_BLOCK - 1)
    def _():
        out_ref[0, 0] = acc_ref[0, 0]


_wmse_tc = pl.pallas_call(
    _wmse_tc_body,
    grid=(TC_ROWS // TC_BLOCK,),
    in_specs=[
        pl.BlockSpec((TC_BLOCK, NCOLS), lambda i: (SC_ROWS // TC_BLOCK + i, 0)),
        pl.BlockSpec((TC_BLOCK, NCOLS), lambda i: (SC_ROWS // TC_BLOCK + i, 0)),
        pl.BlockSpec(memory_space=pltpu.SMEM),
    ],
    out_specs=pl.BlockSpec(memory_space=pltpu.SMEM),
    out_shape=jax.ShapeDtypeStruct((1, 1), jnp.float32),
    scratch_shapes=[pltpu.SMEM((1, 1), jnp.float32)],
)


def kernel(inputs, targets, weight):
    sc_partials = _wmse_sc(inputs, targets, weight)
    tc_partial = _wmse_tc(inputs, targets, weight)
    return (jnp.sum(sc_partials) + tc_partial[0, 0]) / N_ELEMS


# speedup vs baseline: 1.2643x; 1.0016x over previous
import functools

import jax
import jax.numpy as jnp
from jax import lax
from jax.experimental import pallas as pl
from jax.experimental.pallas import tpu as pltpu
from jax.experimental.pallas import tpu_sc as plsc

NROWS, NCOLS = 4096, 2048
N_ELEMS = NROWS * NCOLS
NC, NS, L = 2, 16, 16
NW = NC * NS

SC_ROWS = 2560
TC_ROWS = NROWS - SC_ROWS
TC_BLOCK = 128

ROWS_PER_W = SC_ROWS // NW
RPC = 8
NCH = ROWS_PER_W // RPC
NBUF = 2
NGRP = NCH // NBUF
CSTEPS = NCOLS // L

_mesh = plsc.VectorSubcoreMesh(core_axis_name="c", subcore_axis_name="s")


@functools.partial(
    pl.kernel,
    mesh=_mesh,
    out_type=jax.ShapeDtypeStruct((NW, L), jnp.float32),
    scratch_types=[
        pltpu.VMEM((NBUF, RPC, NCOLS), jnp.float32),
        pltpu.VMEM((NBUF, RPC, NCOLS), jnp.int32),
        pltpu.VMEM((L,), jnp.float32),
        pltpu.VMEM((L,), jnp.float32),
        pltpu.SemaphoreType.DMA,
        pltpu.SemaphoreType.DMA,
        pltpu.SemaphoreType.DMA,
        pltpu.SemaphoreType.DMA,
    ],
)
def _wmse_sc(x_hbm, t_hbm, w_hbm, out_hbm, xbuf, tbuf, wv, accv, *sems):
    semx = sems[:NBUF]
    semt = sems[NBUF:]
    wid = lax.axis_index("s") * NC + lax.axis_index("c")
    base_row = wid * ROWS_PER_W

    pltpu.sync_copy(w_hbm, wv)

    def start(c, b):
        row = base_row + c * RPC
        pltpu.async_copy(x_hbm.at[pl.ds(row, RPC)], xbuf.at[b], semx[b])
        pltpu.async_copy(t_hbm.at[pl.ds(row, RPC)], tbuf.at[b], semt[b])

    def wait(b):
        pltpu.make_async_copy(x_hbm.at[pl.ds(0, RPC)], xbuf.at[b], semx[b]).wait()
        pltpu.make_async_copy(t_hbm.at[pl.ds(0, RPC)], tbuf.at[b], semt[b]).wait()

    for b in range(NBUF):
        start(b, b)

    wtab = wv[...]

    def chunk_compute(b, accs):
        def step(i, accs):
            o = i * L
            accs = list(accs)
            for r in range(RPC):
                t = tbuf[b, r, pl.ds(o, L)]
                x = xbuf[b, r, pl.ds(o, L)]
                w = wtab.at[t].get(mode="promise_in_bounds")
                d = x - t.astype(jnp.float32)
                accs[r % 4] = accs[r % 4] + (w * d) * d
            return tuple(accs)

        return lax.fori_loop(0, CSTEPS, step, accs)

    def group(g, accs):
        for b in range(NBUF):
            c = g * NBUF + b
            wait(b)
            accs = chunk_compute(b, accs)
            start(c + NBUF, b)
        return accs

    zero = jnp.zeros((L,), jnp.float32)
    accs = (zero, zero, zero, zero)
    accs = lax.fori_loop(0, NGRP - 1, group, accs)
    for b in range(NBUF):
        wait(b)
        accs = chunk_compute(b, accs)

    accv[...] = (accs[0] + accs[1]) + (accs[2] + accs[3])
    pltpu.sync_copy(accv, out_hbm.at[wid])


def _wmse_tc_body(x_ref, t_ref, w_ref, out_ref, acc_ref):
    i = pl.program_id(0)

    t = t_ref[...]
    x = x_ref[...]
    d = x - t.astype(jnp.float32)
    s = d * d
    w = jnp.full(t.shape, w_ref[0], jnp.float32)
    for k in range(1, 16):
        w = jnp.where(t == k, w_ref[k], w)
    val = w * s
    val8 = val[0:8, :]
    for j in range(1, TC_BLOCK // 8):
        val8 = val8 + val[8 * j : 8 * (j + 1), :]

    @pl.when(i == 0)
    def _():
        acc_ref[...] = jnp.zeros_like(acc_ref)

    acc_ref[...] += val8

    @pl.when(i == TC_ROWS // TC_BLOCK - 1)
    def _():
        out_ref[0, 0] = jnp.sum(acc_ref[...])


_wmse_tc = pl.pallas_call(
    _wmse_tc_body,
    grid=(TC_ROWS // TC_BLOCK,),
    in_specs=[
        pl.BlockSpec((TC_BLOCK, NCOLS), lambda i: (SC_ROWS // TC_BLOCK + i, 0)),
        pl.BlockSpec((TC_BLOCK, NCOLS), lambda i: (SC_ROWS // TC_BLOCK + i, 0)),
        pl.BlockSpec(memory_space=pltpu.SMEM),
    ],
    out_specs=pl.BlockSpec(memory_space=pltpu.SMEM),
    out_shape=jax.ShapeDtypeStruct((1, 1), jnp.float32),
    scratch_shapes=[pltpu.VMEM((8, NCOLS), jnp.float32)],
)


def kernel(inputs, targets, weight):
    sc_partials = _wmse_sc(inputs, targets, weight)
    tc_partial = _wmse_tc(inputs, targets, weight)
    return (jnp.sum(sc_partials) + tc_partial[0, 0]) / N_ELEMS
